# baseline (speedup 1.0000x reference)
import jax
import jax.numpy as jnp
from jax.experimental import pallas as pl
from jax.experimental.pallas import tpu as pltpu

_N, _K, _D = 8192, 8192, 256
_BN = 1024
_BK = 2048


def _mean_kernel(x_ref, mean_ref):
    mean_ref[...] = jnp.sum(x_ref[...], axis=0, keepdims=True) * (1.0 / _N)


def _var_kernel(x_ref, mean_ref, var_ref):
    d = x_ref[...] - mean_ref[...]
    var_ref[...] = jnp.sum(d * d, axis=0, keepdims=True) * (1.0 / _N)


def _norm_kernel(x_ref, mean_ref, var_ref, xnb_ref, rn_ref):
    xn = (x_ref[...] - mean_ref[...]) / jnp.sqrt(var_ref[...] + 1e-5)
    xnb_ref[...] = (xn * (-2.0)).astype(jnp.bfloat16)
    rn_ref[...] = jnp.sum(xn * xn, axis=1, keepdims=True)


def _argmin_kernel(xnb_ref, rn_ref, e_ref, out_ref,
                   ebf_ref, esq_ref, bv_ref, bi_ref):
    j = pl.program_id(0)
    i = pl.program_id(1)

    @pl.when(i == 0)
    def _():
        e = e_ref[...]
        ebf_ref[...] = e.astype(jnp.bfloat16)
        esq_ref[...] = jnp.sum(e * e, axis=1)[None, :]

    dot = jax.lax.dot_general(
        xnb_ref[...], ebf_ref[...], (((1,), (1,)), ((), ())),
        preferred_element_type=jnp.float32)
    rows = pl.ds(i * _BN, _BN)
    dist = (rn_ref[...] + esq_ref[...]) + dot
    lmin = jnp.min(dist, axis=1, keepdims=True)
    ids = jax.lax.broadcasted_iota(jnp.int32, dist.shape, 1)
    larg = jnp.min(jnp.where(dist == lmin, ids, _BK), axis=1,
                   keepdims=True) + j * _BK

    @pl.when(j == 0)
    def _():
        bv_ref[rows, :] = lmin.astype(jnp.bfloat16).astype(jnp.float32)
        bi_ref[rows, :] = larg

    @pl.when(j > 0)
    def _():
        bv = bv_ref[rows, :]
        upd = lmin < bv
        nv = jnp.where(upd, lmin, bv)
        bv_ref[rows, :] = nv.astype(jnp.bfloat16).astype(jnp.float32)
        bi_ref[rows, :] = jnp.where(upd, larg, bi_ref[rows, :])

    @pl.when(j == pl.num_programs(0) - 1)
    def _():
        out_ref[rows, :] = bi_ref[rows, :]


@jax.jit
def _encode(x, embedding):
    mean = pl.pallas_call(
        _mean_kernel,
        grid=(1,),
        in_specs=[pl.BlockSpec((_N, _D), lambda i: (0, 0))],
        out_specs=pl.BlockSpec((1, _D), lambda i: (0, 0)),
        out_shape=jax.ShapeDtypeStruct((1, _D), jnp.float32),
    )(x)
    var = pl.pallas_call(
        _var_kernel,
        grid=(1,),
        in_specs=[pl.BlockSpec((_N, _D), lambda i: (0, 0)),
                  pl.BlockSpec((1, _D), lambda i: (0, 0))],
        out_specs=pl.BlockSpec((1, _D), lambda i: (0, 0)),
        out_shape=jax.ShapeDtypeStruct((1, _D), jnp.float32),
    )(x, mean)
    xnb, rn = pl.pallas_call(
        _norm_kernel,
        grid=(1,),
        in_specs=[pl.BlockSpec((_N, _D), lambda i: (0, 0)),
                  pl.BlockSpec((1, _D), lambda i: (0, 0)),
                  pl.BlockSpec((1, _D), lambda i: (0, 0))],
        out_specs=[pl.BlockSpec((_N, _D), lambda i: (0, 0)),
                   pl.BlockSpec((_N, 1), lambda i: (0, 0))],
        out_shape=[jax.ShapeDtypeStruct((_N, _D), jnp.bfloat16),
                   jax.ShapeDtypeStruct((_N, 1), jnp.float32)],
    )(x, mean, var)

    idx = pl.pallas_call(
        _argmin_kernel,
        grid=(_K // _BK, _N // _BN),
        in_specs=[pl.BlockSpec((_BN, _D), lambda j, i: (i, 0)),
                  pl.BlockSpec((_BN, 1), lambda j, i: (i, 0)),
                  pl.BlockSpec((_BK, _D), lambda j, i: (j, 0))],
        out_specs=pl.BlockSpec((_N, 1), lambda j, i: (0, 0)),
        out_shape=jax.ShapeDtypeStruct((_N, 1), jnp.int32),
        scratch_shapes=[pltpu.VMEM((_BK, _D), jnp.bfloat16),
                        pltpu.VMEM((1, _BK), jnp.float32),
                        pltpu.VMEM((_N, 1), jnp.float32),
                        pltpu.VMEM((_N, 1), jnp.int32)],
    )(xnb, rn, embedding)
    return idx


def kernel(x, embedding, ema_w, ema_cluster_size, running_mean, running_var):
    return _encode(x, embedding)


# speedup vs baseline: 1.0412x; 1.0412x over previous
import jax
import jax.numpy as jnp
from jax.experimental import pallas as pl
from jax.experimental.pallas import tpu as pltpu

_N, _K, _D = 8192, 8192, 256
_BN = 1024
_BK = 2048
_NK = _K // _BK


def _mean_kernel(x_ref, mean_ref):
    mean_ref[...] = jnp.sum(x_ref[...], axis=0, keepdims=True) * (1.0 / _N)


def _var_kernel(x_ref, mean_ref, var_ref):
    d = x_ref[...] - mean_ref[...]
    var_ref[...] = jnp.sum(d * d, axis=0, keepdims=True) * (1.0 / _N)


def _argmin_kernel(x_ref, e_ref, mean_ref, var_ref, out_ref,
                   xnb_ref, rn_ref, ebf_ref, esq_ref, bv_ref, bi_ref):
    i = pl.program_id(0)
    j = pl.program_id(1)

    @pl.when(j == 0)
    def _():
        xb = x_ref[...]
        xn = (xb - mean_ref[...]) / jnp.sqrt(var_ref[...] + 1e-5)
        xnb_ref[...] = (xn * (-2.0)).astype(jnp.bfloat16)
        rn_ref[...] = jnp.sum(xn * xn, axis=1, keepdims=True)

    @pl.when(i == 0)
    def _():
        e = e_ref[...]
        ebf_ref[j] = e.astype(jnp.bfloat16)
        esq_ref[j] = jnp.sum(e * e, axis=1)[None, :]

    dot = jax.lax.dot_general(
        xnb_ref[...], ebf_ref[j], (((1,), (1,)), ((), ())),
        preferred_element_type=jnp.float32)
    dist = (rn_ref[...] + esq_ref[j]) + dot
    lmin = jnp.min(dist, axis=1, keepdims=True)
    ids = jax.lax.broadcasted_iota(jnp.int32, dist.shape, 1)
    larg = jnp.min(jnp.where(dist == lmin, ids, _BK), axis=1,
                   keepdims=True) + j * _BK

    @pl.when(j == 0)
    def _():
        bv_ref[...] = lmin.astype(jnp.bfloat16).astype(jnp.float32)
        bi_ref[...] = larg

    @pl.when(j > 0)
    def _():
        upd = lmin < bv_ref[...]
        nv = jnp.where(upd, lmin, bv_ref[...])
        bv_ref[...] = nv.astype(jnp.bfloat16).astype(jnp.float32)
        bi_ref[...] = jnp.where(upd, larg, bi_ref[...])

    @pl.when(j == pl.num_programs(1) - 1)
    def _():
        out_ref[pl.ds(i * _BN, _BN), :] = bi_ref[...]


@jax.jit
def _encode(x, embedding):
    mean = pl.pallas_call(
        _mean_kernel,
        grid=(1,),
        in_specs=[pl.BlockSpec((_N, _D), lambda i: (0, 0))],
        out_specs=pl.BlockSpec((1, _D), lambda i: (0, 0)),
        out_shape=jax.ShapeDtypeStruct((1, _D), jnp.float32),
    )(x)
    var = pl.pallas_call(
        _var_kernel,
        grid=(1,),
        in_specs=[pl.BlockSpec((_N, _D), lambda i: (0, 0)),
                  pl.BlockSpec((1, _D), lambda i: (0, 0))],
        out_specs=pl.BlockSpec((1, _D), lambda i: (0, 0)),
        out_shape=jax.ShapeDtypeStruct((1, _D), jnp.float32),
    )(x, mean)

    idx = pl.pallas_call(
        _argmin_kernel,
        grid=(_N // _BN, _NK),
        in_specs=[pl.BlockSpec((_BN, _D), lambda i, j: (i, 0)),
                  pl.BlockSpec((_BK, _D), lambda i, j: (j, 0)),
                  pl.BlockSpec((1, _D), lambda i, j: (0, 0)),
                  pl.BlockSpec((1, _D), lambda i, j: (0, 0))],
        out_specs=pl.BlockSpec((_N, 1), lambda i, j: (0, 0)),
        out_shape=jax.ShapeDtypeStruct((_N, 1), jnp.int32),
        scratch_shapes=[pltpu.VMEM((_BN, _D), jnp.bfloat16),
                        pltpu.VMEM((_BN, 1), jnp.float32),
                        pltpu.VMEM((_NK, _BK, _D), jnp.bfloat16),
                        pltpu.VMEM((_NK, 1, _BK), jnp.float32),
                        pltpu.VMEM((_BN, 1), jnp.float32),
                        pltpu.VMEM((_BN, 1), jnp.int32)],
    )(x, embedding, mean, var)
    return idx


def kernel(x, embedding, ema_w, ema_cluster_size, running_mean, running_var):
    return _encode(x, embedding)


# speedup vs baseline: 1.1221x; 1.0777x over previous
import jax
import jax.numpy as jnp
from jax.experimental import pallas as pl
from jax.experimental.pallas import tpu as pltpu

_N, _K, _D = 8192, 8192, 256
_BN = 1024
_BK = 2048


def _mean_kernel(x_ref, mean_ref):
    mean_ref[...] = jnp.sum(x_ref[...], axis=0, keepdims=True) * (1.0 / _N)


def _var_kernel(x_ref, mean_ref, var_ref):
    d = x_ref[...] - mean_ref[...]
    var_ref[...] = jnp.sum(d * d, axis=0, keepdims=True) * (1.0 / _N)


def _argmin_kernel(x_ref, e_ref, mean_ref, var_ref, out_ref,
                   xnb_ref, rn_ref, bv_ref, bi_ref):
    i = pl.program_id(0)
    j = pl.program_id(1)

    @pl.when(j == 0)
    def _():
        xb = x_ref[...]
        xn = (xb - mean_ref[...]) / jnp.sqrt(var_ref[...] + 1e-5)
        xnb_ref[...] = (xn * (-2.0)).astype(jnp.bfloat16)
        rn_ref[...] = jnp.sum(xn * xn, axis=1, keepdims=True)

    e = e_ref[...]
    esq = jnp.sum(e * e, axis=1)
    dot = jax.lax.dot_general(
        xnb_ref[...], e.astype(jnp.bfloat16), (((1,), (1,)), ((), ())),
        preferred_element_type=jnp.float32)
    dist = (rn_ref[...] + esq[None, :]) + dot
    lmin = jnp.min(dist, axis=1, keepdims=True)
    ids = jax.lax.broadcasted_iota(jnp.int32, dist.shape, 1)
    larg = jnp.min(jnp.where(dist == lmin, ids, _BK), axis=1,
                   keepdims=True) + j * _BK

    @pl.when(j == 0)
    def _():
        bv_ref[...] = lmin.astype(jnp.bfloat16).astype(jnp.float32)
        bi_ref[...] = larg

    @pl.when(j > 0)
    def _():
        upd = lmin < bv_ref[...]
        nv = jnp.where(upd, lmin, bv_ref[...])
        bv_ref[...] = nv.astype(jnp.bfloat16).astype(jnp.float32)
        bi_ref[...] = jnp.where(upd, larg, bi_ref[...])

    @pl.when(j == pl.num_programs(1) - 1)
    def _():
        out_ref[pl.ds(i * _BN, _BN), :] = bi_ref[...]


@jax.jit
def _encode(x, embedding):
    mean = pl.pallas_call(
        _mean_kernel,
        grid=(1,),
        in_specs=[pl.BlockSpec((_N, _D), lambda i: (0, 0))],
        out_specs=pl.BlockSpec((1, _D), lambda i: (0, 0)),
        out_shape=jax.ShapeDtypeStruct((1, _D), jnp.float32),
    )(x)
    var = pl.pallas_call(
        _var_kernel,
        grid=(1,),
        in_specs=[pl.BlockSpec((_N, _D), lambda i: (0, 0)),
                  pl.BlockSpec((1, _D), lambda i: (0, 0))],
        out_specs=pl.BlockSpec((1, _D), lambda i: (0, 0)),
        out_shape=jax.ShapeDtypeStruct((1, _D), jnp.float32),
    )(x, mean)

    idx = pl.pallas_call(
        _argmin_kernel,
        grid=(_N // _BN, _K // _BK),
        in_specs=[pl.BlockSpec((_BN, _D), lambda i, j: (i, 0)),
                  pl.BlockSpec((_BK, _D), lambda i, j: (j, 0)),
                  pl.BlockSpec((1, _D), lambda i, j: (0, 0)),
                  pl.BlockSpec((1, _D), lambda i, j: (0, 0))],
        out_specs=pl.BlockSpec((_N, 1), lambda i, j: (0, 0)),
        out_shape=jax.ShapeDtypeStruct((_N, 1), jnp.int32),
        scratch_shapes=[pltpu.VMEM((_BN, _D), jnp.bfloat16),
                        pltpu.VMEM((_BN, 1), jnp.float32),
                        pltpu.VMEM((_BN, 1), jnp.float32),
                        pltpu.VMEM((_BN, 1), jnp.int32)],
    )(x, embedding, mean, var)
    return idx


def kernel(x, embedding, ema_w, ema_cluster_size, running_mean, running_var):
    return _encode(x, embedding)


# speedup vs baseline: 1.1678x; 1.0408x over previous
import jax
import jax.numpy as jnp
from jax.experimental import pallas as pl
from jax.experimental.pallas import tpu as pltpu

_N, _K, _D = 8192, 8192, 256
_BN = 1024
_BK = 2048
_NJ = _K // _BK


def _vq_kernel(x_ref, e_ref, out_ref,
               mean_ref, var_ref, xnb_ref, rn_ref, bv_ref, bi_ref):
    s = pl.program_id(0)
    i = s // _NJ
    j = s % _NJ

    @pl.when(s == 0)
    def _():
        xf = x_ref[...]
        m = jnp.sum(xf, axis=0, keepdims=True) * (1.0 / _N)
        mean_ref[...] = m
        d = xf - m
        var_ref[...] = jnp.sum(d * d, axis=0, keepdims=True) * (1.0 / _N)

    @pl.when(j == 0)
    def _():
        xb = x_ref[pl.ds(i * _BN, _BN), :]
        xn = (xb - mean_ref[...]) / jnp.sqrt(var_ref[...] + 1e-5)
        xnb_ref[...] = (xn * (-2.0)).astype(jnp.bfloat16)
        rn_ref[...] = jnp.sum(xn * xn, axis=1, keepdims=True)

    e = e_ref[pl.ds(j * _BK, _BK), :]
    esq = jnp.sum(e * e, axis=1)
    dot = jax.lax.dot_general(
        xnb_ref[...], e.astype(jnp.bfloat16), (((1,), (1,)), ((), ())),
        preferred_element_type=jnp.float32)
    dist = (rn_ref[...] + esq[None, :]) + dot
    lmin = jnp.min(dist, axis=1, keepdims=True)
    ids = jax.lax.broadcasted_iota(jnp.int32, dist.shape, 1)
    larg = jnp.min(jnp.where(dist == lmin, ids, _BK), axis=1,
                   keepdims=True) + j * _BK

    @pl.when(j == 0)
    def _():
        bv_ref[...] = lmin.astype(jnp.bfloat16).astype(jnp.float32)
        bi_ref[...] = larg

    @pl.when(j > 0)
    def _():
        upd = lmin < bv_ref[...]
        nv = jnp.where(upd, lmin, bv_ref[...])
        bv_ref[...] = nv.astype(jnp.bfloat16).astype(jnp.float32)
        bi_ref[...] = jnp.where(upd, larg, bi_ref[...])

    @pl.when(j == _NJ - 1)
    def _():
        out_ref[pl.ds(i * _BN, _BN), :] = bi_ref[...]


@jax.jit
def _encode(x, embedding):
    return pl.pallas_call(
        _vq_kernel,
        grid=((_N // _BN) * _NJ,),
        in_specs=[pl.BlockSpec((_N, _D), lambda s: (0, 0)),
                  pl.BlockSpec((_K, _D), lambda s: (0, 0))],
        out_specs=pl.BlockSpec((_N, 1), lambda s: (0, 0)),
        out_shape=jax.ShapeDtypeStruct((_N, 1), jnp.int32),
        scratch_shapes=[pltpu.VMEM((1, _D), jnp.float32),
                        pltpu.VMEM((1, _D), jnp.float32),
                        pltpu.VMEM((_BN, _D), jnp.bfloat16),
                        pltpu.VMEM((_BN, 1), jnp.float32),
                        pltpu.VMEM((_BN, 1), jnp.float32),
                        pltpu.VMEM((_BN, 1), jnp.int32)],
    )(x, embedding)


def kernel(x, embedding, ema_w, ema_cluster_size, running_mean, running_var):
    return _encode(x, embedding)


# speedup vs baseline: 1.2105x; 1.0366x over previous
import jax
import jax.numpy as jnp
from jax.experimental import pallas as pl
from jax.experimental.pallas import tpu as pltpu

_N, _K, _D = 8192, 8192, 256
_BN = 1024
_BK = 2048
_NJ = _K // _BK


def _vq_kernel(x_ref, e_ref, out_ref,
               mean_ref, var_ref, xnb_ref, rn_ref, bv_ref, bi_ref,
               ebf_ref, esq_ref, idsf_ref):
    s = pl.program_id(0)
    i = s // _NJ
    j = s % _NJ

    @pl.when(s == 0)
    def _():
        xf = x_ref[...]
        m = jnp.sum(xf, axis=0, keepdims=True) * (1.0 / _N)
        mean_ref[...] = m
        d = xf - m
        var_ref[...] = jnp.sum(d * d, axis=0, keepdims=True) * (1.0 / _N)
        ef = e_ref[...]
        ebf_ref[...] = ef.astype(jnp.bfloat16)
        esq_ref[...] = jnp.sum(ef * ef, axis=1)[None, :]
        idsf_ref[...] = jax.lax.broadcasted_iota(
            jnp.int32, (1, _BK), 1).astype(jnp.float32)

    @pl.when(j == 0)
    def _():
        xb = x_ref[pl.ds(i * _BN, _BN), :]
        xn = (xb - mean_ref[...]) / jnp.sqrt(var_ref[...] + 1e-5)
        xnb_ref[...] = (xn * (-2.0)).astype(jnp.bfloat16)
        rn_ref[...] = jnp.sum(xn * xn, axis=1, keepdims=True)

    dot = jax.lax.dot_general(
        xnb_ref[...], ebf_ref[pl.ds(j * _BK, _BK), :],
        (((1,), (1,)), ((), ())),
        preferred_element_type=jnp.float32)
    dist = (rn_ref[...] + esq_ref[:, pl.ds(j * _BK, _BK)]) + dot
    lmin = jnp.min(dist, axis=1, keepdims=True)
    larg_f = jnp.min(jnp.where(dist == lmin, idsf_ref[...], float(_BK)),
                     axis=1, keepdims=True)
    larg = larg_f.astype(jnp.int32) + j * _BK

    @pl.when(j == 0)
    def _():
        bv_ref[...] = lmin.astype(jnp.bfloat16).astype(jnp.float32)
        bi_ref[...] = larg

    @pl.when(j > 0)
    def _():
        upd = lmin < bv_ref[...]
        nv = jnp.where(upd, lmin, bv_ref[...])
        bv_ref[...] = nv.astype(jnp.bfloat16).astype(jnp.float32)
        bi_ref[...] = jnp.where(upd, larg, bi_ref[...])

    @pl.when(j == _NJ - 1)
    def _():
        out_ref[pl.ds(i * _BN, _BN), :] = bi_ref[...]


@jax.jit
def _encode(x, embedding):
    return pl.pallas_call(
        _vq_kernel,
        grid=((_N // _BN) * _NJ,),
        in_specs=[pl.BlockSpec((_N, _D), lambda s: (0, 0)),
                  pl.BlockSpec((_K, _D), lambda s: (0, 0))],
        out_specs=pl.BlockSpec((_N, 1), lambda s: (0, 0)),
        out_shape=jax.ShapeDtypeStruct((_N, 1), jnp.int32),
        scratch_shapes=[pltpu.VMEM((1, _D), jnp.float32),
                        pltpu.VMEM((1, _D), jnp.float32),
                        pltpu.VMEM((_BN, _D), jnp.bfloat16),
                        pltpu.VMEM((_BN, 1), jnp.float32),
                        pltpu.VMEM((_BN, 1), jnp.float32),
                        pltpu.VMEM((_BN, 1), jnp.int32),
                        pltpu.VMEM((_K, _D), jnp.bfloat16),
                        pltpu.VMEM((1, _K), jnp.float32),
                        pltpu.VMEM((1, _BK), jnp.float32)],
    )(x, embedding)


def kernel(x, embedding, ema_w, ema_cluster_size, running_mean, running_var):
    return _encode(x, embedding)
